# Initial kernel scaffold; baseline (speedup 1.0000x reference)
import jax
import jax.numpy as jnp
from jax.experimental import pallas as pl


def kernel(coords, table0, table1):
    raise NotImplementedError("write your pallas kernel here")


# speedup vs baseline: 1.3343x; 1.3343x over previous
import functools

import jax
import jax.numpy as jnp
from jax import lax
from jax.experimental import pallas as pl
from jax.experimental.pallas import tpu as pltpu
from jax.experimental.pallas import tpu_sc as plsc

POS_DIM = 1024
B_TOTAL = 4 * 4096
NUM_CORES = 2
NUM_SUBCORES = 16
NW = NUM_CORES * NUM_SUBCORES
B_PER_W = B_TOTAL // NW
CHUNK = 32
N_CHUNKS = B_PER_W // CHUNK
LANES = 16

_mesh = plsc.VectorSubcoreMesh(
    core_axis_name="c", subcore_axis_name="s",
    num_cores=NUM_CORES, num_subcores=NUM_SUBCORES)


@functools.partial(
    pl.kernel,
    out_type=jax.ShapeDtypeStruct((B_TOTAL, POS_DIM), jnp.float32),
    mesh=_mesh,
    scratch_types=[
        pltpu.VMEM((N_CHUNKS, CHUNK), jnp.int32),
        pltpu.VMEM((N_CHUNKS, CHUNK), jnp.int32),
        pltpu.VMEM((CHUNK, POS_DIM), jnp.float32),
        pltpu.VMEM((CHUNK, POS_DIM), jnp.float32),
        pltpu.SemaphoreType.DMA,
        pltpu.SemaphoreType.DMA,
    ],
)
def _embed_sum(c0_hbm, c1_hbm, t0_hbm, t1_hbm, out_hbm,
               idx0_v, idx1_v, buf0, buf1, sem0, sem1):
    wid = lax.axis_index("s") * NUM_CORES + lax.axis_index("c")
    base = wid * B_PER_W
    pltpu.sync_copy(c0_hbm.at[wid], idx0_v)
    pltpu.sync_copy(c1_hbm.at[wid], idx1_v)

    def chunk_body(c, carry):
        cp0 = pltpu.async_copy(t0_hbm.at[idx0_v.at[c]], buf0, sem0)
        cp1 = pltpu.async_copy(t1_hbm.at[idx1_v.at[c]], buf1, sem1)
        cp0.wait()
        cp1.wait()

        def row_body(r, rc):
            for j in range(POS_DIM // LANES):
                plsc.addupdate(buf0.at[r, pl.ds(j * LANES, LANES)],
                               buf1[r, pl.ds(j * LANES, LANES)])
            return rc
        lax.fori_loop(0, CHUNK, row_body, 0)

        pltpu.sync_copy(buf0, out_hbm.at[pl.ds(base + c * CHUNK, CHUNK)])
        return carry

    lax.fori_loop(0, N_CHUNKS, chunk_body, 0)


def kernel(coords, table0, table1):
    c = coords.reshape(2, NW, N_CHUNKS, CHUNK)
    out = _embed_sum(c[0], c[1], table0, table1)
    return out.reshape(4, 4096, POS_DIM)


# speedup vs baseline: 1.9771x; 1.4818x over previous
import functools

import jax
import jax.numpy as jnp
from jax import lax
from jax.experimental import pallas as pl
from jax.experimental.pallas import tpu as pltpu
from jax.experimental.pallas import tpu_sc as plsc

POS_DIM = 1024
B_TOTAL = 4 * 4096
NUM_CORES = 2
NUM_SUBCORES = 16
NW = NUM_CORES * NUM_SUBCORES
B_PER_W = B_TOTAL // NW
CHUNK = 16
N_CHUNKS = B_PER_W // CHUNK
LANES = 16

_mesh = plsc.VectorSubcoreMesh(
    core_axis_name="c", subcore_axis_name="s",
    num_cores=NUM_CORES, num_subcores=NUM_SUBCORES)


@functools.partial(
    pl.kernel,
    out_type=jax.ShapeDtypeStruct((B_TOTAL, POS_DIM), jnp.float32),
    mesh=_mesh,
    scratch_types=[
        pltpu.VMEM((N_CHUNKS, CHUNK), jnp.int32),
        pltpu.VMEM((N_CHUNKS, CHUNK), jnp.int32),
        pltpu.VMEM((CHUNK, POS_DIM), jnp.float32),
        pltpu.VMEM((CHUNK, POS_DIM), jnp.float32),
        pltpu.VMEM((CHUNK, POS_DIM), jnp.float32),
        pltpu.VMEM((CHUNK, POS_DIM), jnp.float32),
        pltpu.SemaphoreType.DMA,
        pltpu.SemaphoreType.DMA,
        pltpu.SemaphoreType.DMA,
        pltpu.SemaphoreType.DMA,
    ],
)
def _embed_sum(c0_hbm, c1_hbm, t0_hbm, t1_hbm, out_hbm,
               idx0_v, idx1_v, a0, a1, b0, b1, sga, sgb, soa, sob):
    wid = lax.axis_index("s") * NUM_CORES + lax.axis_index("c")
    base = wid * B_PER_W
    pltpu.sync_copy(c0_hbm.at[wid], idx0_v)
    pltpu.sync_copy(c1_hbm.at[wid], idx1_v)

    def issue_gathers(c, d0, d1, sem):
        pltpu.async_copy(t0_hbm.at[idx0_v.at[c]], d0, sem)
        pltpu.async_copy(t1_hbm.at[idx1_v.at[c]], d1, sem)

    def wait_gathers(d0, d1, sem):
        pltpu.make_async_copy(t0_hbm.at[idx0_v.at[0]], d0, sem).wait()
        pltpu.make_async_copy(t1_hbm.at[idx1_v.at[0]], d1, sem).wait()

    def accumulate(d0, d1):
        def row_body(r, rc):
            for j in range(POS_DIM // LANES):
                plsc.addupdate(d0.at[r, pl.ds(j * LANES, LANES)],
                               d1[r, pl.ds(j * LANES, LANES)])
            return rc
        lax.fori_loop(0, CHUNK, row_body, 0)

    def start_out(c, d0, sem):
        pltpu.async_copy(d0, out_hbm.at[pl.ds(base + c * CHUNK, CHUNK)], sem)

    def wait_out(d0, sem):
        pltpu.make_async_copy(d0, out_hbm.at[pl.ds(0, CHUNK)], sem).wait()

    issue_gathers(0, a0, a1, sga)
    issue_gathers(1, b0, b1, sgb)

    def body(k, carry):
        ca = 2 * k
        cb = 2 * k + 1
        wait_gathers(a0, a1, sga)
        accumulate(a0, a1)
        start_out(ca, a0, soa)
        wait_gathers(b0, b1, sgb)
        accumulate(b0, b1)
        start_out(cb, b0, sob)
        wait_out(a0, soa)
        issue_gathers(ca + 2, a0, a1, sga)
        wait_out(b0, sob)
        issue_gathers(cb + 2, b0, b1, sgb)
        return carry

    lax.fori_loop(0, N_CHUNKS // 2 - 1, body, 0)

    wait_gathers(a0, a1, sga)
    accumulate(a0, a1)
    start_out(N_CHUNKS - 2, a0, soa)
    wait_gathers(b0, b1, sgb)
    accumulate(b0, b1)
    start_out(N_CHUNKS - 1, b0, sob)
    wait_out(a0, soa)
    wait_out(b0, sob)


def kernel(coords, table0, table1):
    c = coords.reshape(2, NW, N_CHUNKS, CHUNK)
    out = _embed_sum(c[0], c[1], table0, table1)
    return out.reshape(4, 4096, POS_DIM)
